# Initial kernel scaffold; baseline (speedup 1.0000x reference)
import jax
import jax.numpy as jnp
from jax.experimental import pallas as pl


def kernel(x, edge_index, batch, W1, b1, W2, b2, W3, b3, Wf1, bf1, Wf2, bf2):
    raise NotImplementedError("write your pallas kernel here")


# speedup vs baseline: 11.3074x; 11.3074x over previous
import functools

import jax
import jax.numpy as jnp
from jax import lax
from jax.experimental import pallas as pl
from jax.experimental.pallas import tpu as pltpu
from jax.experimental.pallas import tpu_sc as plsc

_N = 10000
_E = 320000
_G = 64
_H = 32
_NC = 2
_NS = 16
_NW = _NC * _NS
_CHUNK = 128
_NCH = 80
_EPW = _NCH * _CHUNK
_EPAD = _NW * _EPW
_RPS = 640
_NP = _NS * _RPS

_F32 = jnp.float32
_HIGH = lax.Precision.HIGHEST


def _mesh():
    return plsc.VectorSubcoreMesh(core_axis_name="c", subcore_axis_name="s")


def _deg_body(dst_hbm, ones_hbm, zrow_hbm, out_hbm, dst_v, ones_v, acc_sh):
    cid = lax.axis_index("c")
    sid = lax.axis_index("s")
    wid = sid * _NC + cid
    base = sid * _RPS
    pltpu.sync_copy(zrow_hbm, acc_sh.at[pl.ds(base, _RPS)])
    pltpu.sync_copy(ones_hbm, ones_v)
    pltpu.sync_copy(dst_hbm.at[wid], dst_v)
    plsc.subcore_barrier()

    def step(j, carry):
        pltpu.sync_copy(ones_v, acc_sh.at[dst_v.at[j]], add=True)
        return carry

    lax.fori_loop(0, _NCH, step, 0)
    plsc.subcore_barrier()
    pltpu.sync_copy(acc_sh.at[pl.ds(base, _RPS)],
                    out_hbm.at[cid].at[pl.ds(base, _RPS)])


_deg_kernel = pl.kernel(
    _deg_body,
    out_type=jax.ShapeDtypeStruct((_NC, _NP), _F32),
    mesh=_mesh(),
    scratch_types=[
        pltpu.VMEM((_NCH, _CHUNK), jnp.int32),
        pltpu.VMEM((_CHUNK,), _F32),
        pltpu.VMEM_SHARED((_NP,), _F32),
    ],
)


def _make_prop(width):
    def body(u_hbm, src_hbm, dst_hbm, zrow_hbm, out_hbm,
             src_v, dst_v, rows_v, acc_sh, gsem0, gsem1):
        cid = lax.axis_index("c")
        sid = lax.axis_index("s")
        wid = sid * _NC + cid
        base = sid * _RPS
        pltpu.sync_copy(zrow_hbm, acc_sh.at[pl.ds(base, _RPS)])
        pltpu.sync_copy(src_hbm.at[wid], src_v)
        pltpu.sync_copy(dst_hbm.at[wid], dst_v)
        plsc.subcore_barrier()

        sems = (gsem0, gsem1)
        pltpu.async_copy(u_hbm.at[src_v.at[0]], rows_v.at[0], gsem0)
        pltpu.async_copy(u_hbm.at[src_v.at[1]], rows_v.at[1], gsem1)

        def step(i, carry):
            for b in range(2):
                jj = 2 * i + b
                pltpu.make_async_copy(u_hbm.at[src_v.at[jj]], rows_v.at[b],
                                      sems[b]).wait()
                pltpu.sync_copy(rows_v.at[b], acc_sh.at[dst_v.at[jj]],
                                add=True)

                @pl.when(jj + 2 < _NCH)
                def _():
                    pltpu.async_copy(u_hbm.at[src_v.at[jj + 2]], rows_v.at[b],
                                     sems[b])
            return carry

        lax.fori_loop(0, _NCH // 2, step, 0)
        plsc.subcore_barrier()
        pltpu.sync_copy(acc_sh.at[pl.ds(base, _RPS)],
                        out_hbm.at[cid].at[pl.ds(base, _RPS)])

    return pl.kernel(
        body,
        out_type=jax.ShapeDtypeStruct((_NC, _NP, width), _F32),
        mesh=_mesh(),
        compiler_params=pltpu.CompilerParams(use_tc_tiling_on_sc=False),
        scratch_types=[
            pltpu.VMEM((_NCH, _CHUNK), jnp.int32),
            pltpu.VMEM((_NCH, _CHUNK), jnp.int32),
            pltpu.VMEM((2, _CHUNK, width), _F32),
            pltpu.VMEM_SHARED((_NP, width), _F32),
            pltpu.SemaphoreType.DMA,
            pltpu.SemaphoreType.DMA,
        ],
    )


_prop64 = _make_prop(2 * _H)
_prop32 = _make_prop(_H)


def _dot(a, b):
    return jnp.dot(a, b, preferred_element_type=_F32, precision=_HIGH)


def _tc1_body(degT_ref, x_ref, w1_ref, dinv_ref, c0_ref, uab_ref):
    deg = degT_ref[:, 0:1] + degT_ref[:, 1:2]
    dinv = jnp.where(deg > 0.0, lax.rsqrt(jnp.maximum(deg, 1e-12)), 0.0)
    rows = lax.broadcasted_iota(jnp.int32, (_NP, 1), 0)
    dinv = jnp.where(rows < _N, dinv, 0.0)
    dinv_ref[...] = dinv
    x = x_ref[...]
    c0_ref[...] = _dot(x, w1_ref[0] - w1_ref[2])
    wb = jnp.concatenate([w1_ref[1], w1_ref[2]], axis=1)
    ab = _dot(x, wb)
    uab_ref[0:_N, :] = dinv[0:_N, :] * ab
    uab_ref[_N:, :] = jnp.zeros((_NP - _N, 2 * _H), _F32)


_tc1 = pl.pallas_call(
    _tc1_body,
    out_shape=[
        jax.ShapeDtypeStruct((_NP, 1), _F32),
        jax.ShapeDtypeStruct((_N, _H), _F32),
        jax.ShapeDtypeStruct((_NP, 2 * _H), _F32),
    ],
)


def _tc2_body(q_ref, dinv_ref, pa_ref, upb_ref):
    s = q_ref[0] + q_ref[1]
    dinv = dinv_ref[...]
    pa_ref[...] = (-dinv * s[:, 0:_H])[0:_N]
    upb_ref[...] = -(dinv * dinv) * s[:, _H:2 * _H]


_tc2 = pl.pallas_call(
    _tc2_body,
    out_shape=[
        jax.ShapeDtypeStruct((_N, _H), _F32),
        jax.ShapeDtypeStruct((_NP, _H), _F32),
    ],
)


def _tc3_body(q_ref, dinv_ref, c0_ref, pa_ref, b1_ref, w2_ref, u1_ref, d2_ref):
    s = q_ref[0] + q_ref[1]
    dinv = dinv_ref[...]
    ppb = (-dinv * s)[0:_N]
    h1 = jnp.maximum(c0_ref[...] + pa_ref[...] + 2.0 * ppb + b1_ref[...], 0.0)
    u1_ref[0:_N, :] = dinv[0:_N] * h1
    u1_ref[_N:, :] = jnp.zeros((_NP - _N, _H), _F32)
    d2_ref[...] = _dot(h1, w2_ref[0] - w2_ref[2])


_tc3 = pl.pallas_call(
    _tc3_body,
    out_shape=[
        jax.ShapeDtypeStruct((_NP, _H), _F32),
        jax.ShapeDtypeStruct((_N, _H), _F32),
    ],
)


def _tc4_body(q_ref, dinv_ref, w_ref, e_ref, up_ref):
    s = q_ref[0] + q_ref[1]
    dinv = dinv_ref[...]
    p1 = (-dinv * s)[0:_N]
    e_ref[...] = _dot(p1, w_ref[1])
    up_ref[...] = -(dinv * dinv) * s


_tc4 = pl.pallas_call(
    _tc4_body,
    out_shape=[
        jax.ShapeDtypeStruct((_N, _H), _F32),
        jax.ShapeDtypeStruct((_NP, _H), _F32),
    ],
)


def _tc5_body(q_ref, dinv_ref, d_ref, e_ref, b_ref, w_ref, wn_ref,
              u_ref, dn_ref):
    s = q_ref[0] + q_ref[1]
    dinv = dinv_ref[...]
    p2 = (-dinv * s)[0:_N]
    h = jnp.maximum(
        d_ref[...] + e_ref[...] + 2.0 * _dot(p2, w_ref[2]) + b_ref[...], 0.0)
    u_ref[0:_N, :] = dinv[0:_N] * h
    u_ref[_N:, :] = jnp.zeros((_NP - _N, _H), _F32)
    dn_ref[...] = _dot(h, wn_ref[0] - wn_ref[2])


_tc5 = pl.pallas_call(
    _tc5_body,
    out_shape=[
        jax.ShapeDtypeStruct((_NP, _H), _F32),
        jax.ShapeDtypeStruct((_N, _H), _F32),
    ],
)


def _tc7_body(q_ref, dinv_ref, d_ref, e_ref, b3_ref, w3_ref, batch_ref,
              wf1_ref, bf1_ref, wf2_ref, bf2_ref, out_ref):
    s = q_ref[0] + q_ref[1]
    dinv = dinv_ref[...]
    p2 = (-dinv * s)[0:_N]
    h = jnp.maximum(
        d_ref[...] + e_ref[...] + 2.0 * _dot(p2, w3_ref[2]) + b3_ref[...], 0.0)
    m = (batch_ref[...] ==
         lax.broadcasted_iota(jnp.int32, (_N, _G), 1)).astype(_F32)
    dims = (((0,), (0,)), ((), ()))
    sums = lax.dot_general(m, h, dims, preferred_element_type=_F32,
                           precision=_HIGH)
    cnt = lax.dot_general(m, jnp.ones((_N, 1), _F32), dims,
                          preferred_element_type=_F32, precision=_HIGH)
    pooled = sums / jnp.maximum(cnt, 1.0)
    r = jnp.maximum(_dot(pooled, wf1_ref[...]) + bf1_ref[...], 0.0)
    out_ref[...] = _dot(r, wf2_ref[...]) + bf2_ref[...]


_tc7 = pl.pallas_call(
    _tc7_body,
    out_shape=jax.ShapeDtypeStruct((_G, 1), _F32),
)


def kernel(x, edge_index, batch, W1, b1, W2, b2, W3, b3, Wf1, bf1, Wf2, bf2):
    pad = jnp.full((_EPAD - _E,), _N, jnp.int32)
    srcw = jnp.concatenate([edge_index[0], pad]).reshape(_NW, _NCH, _CHUNK)
    dstw = jnp.concatenate([edge_index[1], pad]).reshape(_NW, _NCH, _CHUNK)

    ones_c = jnp.ones((_CHUNK,), _F32)
    zrow1 = jnp.zeros((_RPS,), _F32)
    zrow32 = jnp.zeros((_RPS, _H), _F32)
    zrow64 = jnp.zeros((_RPS, 2 * _H), _F32)

    degp = _deg_kernel(dstw, ones_c, zrow1)
    dinv, c0, uab = _tc1(degp.T, x, W1)
    qab = _prop64(uab, srcw, dstw, zrow64)
    pa, upb = _tc2(qab, dinv)
    qpb = _prop32(upb, srcw, dstw, zrow32)
    u1, d2 = _tc3(qpb, dinv, c0, pa, b1.reshape(1, _H), W2)
    q1 = _prop32(u1, srcw, dstw, zrow32)
    e2, up1 = _tc4(q1, dinv, W2)
    q2 = _prop32(up1, srcw, dstw, zrow32)
    u2, d3 = _tc5(q2, dinv, d2, e2, b2.reshape(1, _H), W2, W3)
    q1b = _prop32(u2, srcw, dstw, zrow32)
    e3, up2 = _tc4(q1b, dinv, W3)
    q2b = _prop32(up2, srcw, dstw, zrow32)
    out = _tc7(q2b, dinv, d3, e3, b3.reshape(1, _H), W3,
               batch.reshape(_N, 1), Wf1, bf1.reshape(1, _H),
               Wf2, bf2.reshape(1, 1))
    return out
